# Initial kernel scaffold; baseline (speedup 1.0000x reference)
import jax
import jax.numpy as jnp
from jax.experimental import pallas as pl


def kernel(feat0, feat1, token0, token1, mask_nodes0, keep_nodes0, mask_nodes1, keep_nodes1):
    raise NotImplementedError("write your pallas kernel here")


# speedup vs baseline: 5.4754x; 5.4754x over previous
import functools

import jax
import jax.numpy as jnp
from jax import lax
from jax.experimental import pallas as pl
from jax.experimental.pallas import tpu as pltpu
from jax.experimental.pallas import tpu_sc as plsc

N0, D0 = 100000, 256
N1, D1 = 50000, 128
NM0 = N0 // 10
NM1 = N1 // 10

NC, NS, L = 2, 16, 16
NW = NC * NS


def _ceil_mult(x, m):
    return ((x + m - 1) // m) * m


C0 = _ceil_mult(-(-N0 // NW), L)
C1 = _ceil_mult(-(-N1 // NW), L)
NM0P = _ceil_mult(NM0, L)
NM1P = _ceil_mult(NM1, L)

_SENTINEL = 1 << 30


@functools.lru_cache(maxsize=None)
def _mask_build_kernel():
    mesh = plsc.VectorSubcoreMesh(
        core_axis_name="c", subcore_axis_name="s", num_cores=NC, num_subcores=NS
    )
    return pl.kernel(
        _mask_body,
        out_type=(
            jax.ShapeDtypeStruct((NW, C0), jnp.float32),
            jax.ShapeDtypeStruct((NW, C1), jnp.float32),
        ),
        mesh=mesh,
        compiler_params=pltpu.CompilerParams(needs_layout_passes=False),
        scratch_types=[
            pltpu.VMEM((NM0P,), jnp.int32),
            pltpu.VMEM((NM1P,), jnp.int32),
            pltpu.VMEM((C0,), jnp.float32),
            pltpu.VMEM((C1,), jnp.float32),
        ],
    )


def _mask_body(idx0_hbm, idx1_hbm, m0_hbm, m1_hbm, idx0_v, idx1_v, buf0, buf1):
    wid = lax.axis_index("s") * NC + lax.axis_index("c")
    zeros = jnp.zeros((L,), jnp.float32)
    ones = jnp.ones((L,), jnp.float32)

    def zero_chunk(buf, n):
        def body(i, carry):
            buf[pl.ds(i * L, L)] = zeros
            return carry

        lax.fori_loop(0, n // L, body, 0)

    zero_chunk(buf0, C0)
    zero_chunk(buf1, C1)

    pltpu.sync_copy(idx0_hbm, idx0_v)
    pltpu.sync_copy(idx1_hbm, idx1_v)

    def scan_scatter(idx_v, buf, nvec, chunk):
        base = (wid * chunk).astype(jnp.int32)

        def body(j, carry):
            iv = idx_v[pl.ds(j * L, L)]
            rel = iv - base
            msk = (rel >= 0) & (rel < chunk)
            safe = jnp.where(msk, rel, 0)
            plsc.store_scatter(buf, [safe], ones, mask=msk)
            return carry

        lax.fori_loop(0, nvec, body, 0)

    scan_scatter(idx0_v, buf0, NM0P // L, C0)
    scan_scatter(idx1_v, buf1, NM1P // L, C1)

    pltpu.sync_copy(buf0, m0_hbm.at[wid])
    pltpu.sync_copy(buf1, m1_hbm.at[wid])


def _blend_body(feat_ref, m_ref, tok_ref, out_ref):
    out_ref[...] = jnp.where(m_ref[...] > 0.0, tok_ref[...], feat_ref[...])


def _blend(feat, m2, tok, block):
    n, d = feat.shape
    return pl.pallas_call(
        _blend_body,
        grid=(n // block,),
        in_specs=[
            pl.BlockSpec((block, d), lambda i: (i, 0)),
            pl.BlockSpec((block, 1), lambda i: (i, 0)),
            pl.BlockSpec((1, d), lambda i: (0, 0)),
        ],
        out_specs=pl.BlockSpec((block, d), lambda i: (i, 0)),
        out_shape=jax.ShapeDtypeStruct((n, d), jnp.float32),
    )(feat, m2, tok)


def _pad16(idx):
    n = idx.shape[0]
    p = (-n) % L
    if p:
        idx = jnp.concatenate([idx, jnp.full((p,), _SENTINEL, jnp.int32)])
    return idx


def kernel(feat0, feat1, token0, token1, mask_nodes0, keep_nodes0,
           mask_nodes1, keep_nodes1):
    m0, m1 = _mask_build_kernel()(_pad16(mask_nodes0), _pad16(mask_nodes1))
    masked0 = _blend(feat0, m0.reshape(-1, 1), token0, 2000)
    masked1 = _blend(feat1, m1.reshape(-1, 1), token1, 2000)
    return (masked0, masked1, mask_nodes0, keep_nodes0, mask_nodes1, keep_nodes1)


# speedup vs baseline: 5.7438x; 1.0490x over previous
import functools

import jax
import jax.numpy as jnp
from jax import lax
from jax.experimental import pallas as pl
from jax.experimental.pallas import tpu as pltpu
from jax.experimental.pallas import tpu_sc as plsc

N0, D0 = 100000, 256
N1, D1 = 50000, 128
NM0 = N0 // 10
NM1 = N1 // 10

NC, NS, L = 2, 16, 16
NW = NC * NS


def _ceil_mult(x, m):
    return ((x + m - 1) // m) * m


C0 = _ceil_mult(-(-N0 // NW), L)
C1 = _ceil_mult(-(-N1 // NW), L)
NM0P = _ceil_mult(NM0, L)
NM1P = _ceil_mult(NM1, L)

_SENTINEL = 1 << 30


@functools.lru_cache(maxsize=None)
def _mask_build_kernel():
    mesh = plsc.VectorSubcoreMesh(
        core_axis_name="c", subcore_axis_name="s", num_cores=NC, num_subcores=NS
    )
    return pl.kernel(
        _mask_body,
        out_type=(
            jax.ShapeDtypeStruct((NW, C0), jnp.float32),
            jax.ShapeDtypeStruct((NW, C1), jnp.float32),
        ),
        mesh=mesh,
        compiler_params=pltpu.CompilerParams(needs_layout_passes=False),
        scratch_types=[
            pltpu.VMEM((NM0P,), jnp.int32),
            pltpu.VMEM((NM1P,), jnp.int32),
            pltpu.VMEM((C0,), jnp.float32),
            pltpu.VMEM((C1,), jnp.float32),
        ],
    )


def _mask_body(idx0_hbm, idx1_hbm, m0_hbm, m1_hbm, idx0_v, idx1_v, buf0, buf1):
    wid = lax.axis_index("s") * NC + lax.axis_index("c")
    zeros = jnp.zeros((L,), jnp.float32)
    ones = jnp.ones((L,), jnp.float32)

    def zero_chunk(buf, n):
        def body(i, carry):
            buf[pl.ds(i * L, L)] = zeros
            return carry

        lax.fori_loop(0, n // L, body, 0)

    zero_chunk(buf0, C0)
    zero_chunk(buf1, C1)

    pltpu.sync_copy(idx0_hbm, idx0_v)
    pltpu.sync_copy(idx1_hbm, idx1_v)

    def scan_scatter(idx_v, buf, nvec, chunk):
        base = (wid * chunk).astype(jnp.int32)

        def body(j, carry):
            iv = idx_v[pl.ds(j * L, L)]
            rel = iv - base
            msk = (rel >= 0) & (rel < chunk)
            safe = jnp.where(msk, rel, 0)
            plsc.store_scatter(buf, [safe], ones, mask=msk)
            return carry

        lax.fori_loop(0, nvec, body, 0)

    scan_scatter(idx0_v, buf0, NM0P // L, C0)
    scan_scatter(idx1_v, buf1, NM1P // L, C1)

    pltpu.sync_copy(buf0, m0_hbm.at[wid])
    pltpu.sync_copy(buf1, m1_hbm.at[wid])


def _blend_body(feat_ref, m_ref, tok_ref, out_ref):
    out_ref[...] = jnp.where(m_ref[...] > 0.0, tok_ref[...], feat_ref[...])


def _blend(feat, m2, tok, block):
    n, d = feat.shape
    return pl.pallas_call(
        _blend_body,
        grid=(n // block,),
        in_specs=[
            pl.BlockSpec((block, d), lambda i: (i, 0)),
            pl.BlockSpec((block, 1), lambda i: (i, 0)),
            pl.BlockSpec((1, d), lambda i: (0, 0)),
        ],
        out_specs=pl.BlockSpec((block, d), lambda i: (i, 0)),
        out_shape=jax.ShapeDtypeStruct((n, d), jnp.float32),
    )(feat, m2, tok)


def _pad16(idx):
    n = idx.shape[0]
    p = (-n) % L
    if p:
        idx = jnp.concatenate([idx, jnp.full((p,), _SENTINEL, jnp.int32)])
    return idx


def kernel(feat0, feat1, token0, token1, mask_nodes0, keep_nodes0,
           mask_nodes1, keep_nodes1):
    m0, m1 = _mask_build_kernel()(_pad16(mask_nodes0), _pad16(mask_nodes1))
    masked0 = _blend(feat0, m0.reshape(-1, 1), token0, 5000)
    masked1 = _blend(feat1, m1.reshape(-1, 1), token1, 5000)
    return (masked0, masked1, mask_nodes0, keep_nodes0, mask_nodes1, keep_nodes1)


# speedup vs baseline: 5.7698x; 1.0045x over previous
import functools

import jax
import jax.numpy as jnp
from jax import lax
from jax.experimental import pallas as pl
from jax.experimental.pallas import tpu as pltpu
from jax.experimental.pallas import tpu_sc as plsc

N0, D0 = 100000, 256
N1, D1 = 50000, 128
NM0 = N0 // 10
NM1 = N1 // 10

NC, NS, L = 2, 16, 16
NW = NC * NS


def _ceil_mult(x, m):
    return ((x + m - 1) // m) * m


C0 = _ceil_mult(-(-N0 // NW), L)
C1 = _ceil_mult(-(-N1 // NW), L)
NM0P = _ceil_mult(NM0, L)
NM1P = _ceil_mult(NM1, L)

_SENTINEL = 1 << 30


@functools.lru_cache(maxsize=None)
def _mask_build_kernel():
    mesh = plsc.VectorSubcoreMesh(
        core_axis_name="c", subcore_axis_name="s", num_cores=NC, num_subcores=NS
    )
    return pl.kernel(
        _mask_body,
        out_type=(
            jax.ShapeDtypeStruct((NW, C0), jnp.float32),
            jax.ShapeDtypeStruct((NW, C1), jnp.float32),
        ),
        mesh=mesh,
        compiler_params=pltpu.CompilerParams(needs_layout_passes=False),
        scratch_types=[
            pltpu.VMEM((NM0P,), jnp.int32),
            pltpu.VMEM((NM1P,), jnp.int32),
            pltpu.VMEM((C0,), jnp.float32),
            pltpu.VMEM((C1,), jnp.float32),
        ],
    )


def _mask_body(idx0_hbm, idx1_hbm, m0_hbm, m1_hbm, idx0_v, idx1_v, buf0, buf1):
    wid = lax.axis_index("s") * NC + lax.axis_index("c")
    zeros = jnp.zeros((L,), jnp.float32)
    ones = jnp.ones((L,), jnp.float32)

    def zero_chunk(buf, n):
        def body(i, carry):
            buf[pl.ds(i * L, L)] = zeros
            return carry

        lax.fori_loop(0, n // L, body, 0)

    zero_chunk(buf0, C0)
    zero_chunk(buf1, C1)

    pltpu.sync_copy(idx0_hbm, idx0_v)
    pltpu.sync_copy(idx1_hbm, idx1_v)

    def scan_scatter(idx_v, buf, nvec, chunk):
        base = (wid * chunk).astype(jnp.int32)

        def body(j, carry):
            iv = idx_v[pl.ds(j * L, L)]
            rel = iv - base
            msk = (rel >= 0) & (rel < chunk)
            safe = jnp.where(msk, rel, 0)
            plsc.store_scatter(buf, [safe], ones, mask=msk)
            return carry

        lax.fori_loop(0, nvec, body, 0)

    scan_scatter(idx0_v, buf0, NM0P // L, C0)
    scan_scatter(idx1_v, buf1, NM1P // L, C1)

    pltpu.sync_copy(buf0, m0_hbm.at[wid])
    pltpu.sync_copy(buf1, m1_hbm.at[wid])


def _blend_body(feat_ref, m_ref, tok_ref, out_ref):
    out_ref[...] = jnp.where(m_ref[...] > 0.0, tok_ref[...], feat_ref[...])


def _blend(feat, m2, tok, block):
    n, d = feat.shape
    return pl.pallas_call(
        _blend_body,
        grid=(n // block,),
        in_specs=[
            pl.BlockSpec((block, d), lambda i: (i, 0)),
            pl.BlockSpec((block, 1), lambda i: (i, 0)),
            pl.BlockSpec((1, d), lambda i: (0, 0)),
        ],
        out_specs=pl.BlockSpec((block, d), lambda i: (i, 0)),
        out_shape=jax.ShapeDtypeStruct((n, d), jnp.float32),
    )(feat, m2, tok)


def _pad16(idx):
    n = idx.shape[0]
    p = (-n) % L
    if p:
        idx = jnp.concatenate([idx, jnp.full((p,), _SENTINEL, jnp.int32)])
    return idx


def kernel(feat0, feat1, token0, token1, mask_nodes0, keep_nodes0,
           mask_nodes1, keep_nodes1):
    m0, m1 = _mask_build_kernel()(_pad16(mask_nodes0), _pad16(mask_nodes1))
    masked0 = _blend(feat0, m0.reshape(-1, 1), token0, 10000)
    masked1 = _blend(feat1, m1.reshape(-1, 1), token1, 10000)
    return (masked0, masked1, mask_nodes0, keep_nodes0, mask_nodes1, keep_nodes1)


# speedup vs baseline: 6.5632x; 1.1375x over previous
import functools

import jax
import jax.numpy as jnp
from jax import lax
from jax.experimental import pallas as pl
from jax.experimental.pallas import tpu as pltpu
from jax.experimental.pallas import tpu_sc as plsc

N0, D0 = 100000, 256
N1, D1 = 50000, 128
NM0 = N0 // 10
NM1 = N1 // 10

NC, NS, L = 1, 16, 16
NW = NC * NS


def _ceil_mult(x, m):
    return ((x + m - 1) // m) * m


C0 = _ceil_mult(-(-N0 // NW), L)
C1 = _ceil_mult(-(-N1 // NW), L)
NM0P = _ceil_mult(NM0, L)
NM1P = _ceil_mult(NM1, L)

_SENTINEL = 1 << 30


@functools.lru_cache(maxsize=None)
def _mask_build_kernel():
    mesh = plsc.VectorSubcoreMesh(
        core_axis_name="c", subcore_axis_name="s", num_cores=NC, num_subcores=NS
    )
    return pl.kernel(
        _mask_body,
        out_type=(
            jax.ShapeDtypeStruct((NW, C0), jnp.float32),
            jax.ShapeDtypeStruct((NW, C1), jnp.float32),
        ),
        mesh=mesh,
        compiler_params=pltpu.CompilerParams(needs_layout_passes=False),
        scratch_types=[
            pltpu.VMEM((NM0P,), jnp.int32),
            pltpu.VMEM((NM1P,), jnp.int32),
            pltpu.VMEM((C0,), jnp.float32),
            pltpu.VMEM((C1,), jnp.float32),
        ],
    )


def _mask_body(idx0_hbm, idx1_hbm, m0_hbm, m1_hbm, idx0_v, idx1_v, buf0, buf1):
    wid = lax.axis_index("s") * NC + lax.axis_index("c")
    zeros = jnp.zeros((L,), jnp.float32)
    ones = jnp.ones((L,), jnp.float32)

    def zero_chunk(buf, n):
        def body(i, carry):
            buf[pl.ds(i * L, L)] = zeros
            return carry

        lax.fori_loop(0, n // L, body, 0)

    zero_chunk(buf0, C0)
    zero_chunk(buf1, C1)

    pltpu.sync_copy(idx0_hbm, idx0_v)
    pltpu.sync_copy(idx1_hbm, idx1_v)

    def scan_scatter(idx_v, buf, nvec, chunk):
        base = (wid * chunk).astype(jnp.int32)

        def body(j, carry):
            iv = idx_v[pl.ds(j * L, L)]
            rel = iv - base
            msk = (rel >= 0) & (rel < chunk)
            safe = jnp.where(msk, rel, 0)
            plsc.store_scatter(buf, [safe], ones, mask=msk)
            return carry

        lax.fori_loop(0, nvec, body, 0)

    scan_scatter(idx0_v, buf0, NM0P // L, C0)
    scan_scatter(idx1_v, buf1, NM1P // L, C1)

    pltpu.sync_copy(buf0, m0_hbm.at[wid])
    pltpu.sync_copy(buf1, m1_hbm.at[wid])


def _blend_body(feat_ref, m_ref, tok_ref, out_ref):
    out_ref[...] = jnp.where(m_ref[...] > 0.0, tok_ref[...], feat_ref[...])


def _blend(feat, m2, tok, block):
    n, d = feat.shape
    return pl.pallas_call(
        _blend_body,
        grid=(n // block,),
        in_specs=[
            pl.BlockSpec((block, d), lambda i: (i, 0)),
            pl.BlockSpec((block, 1), lambda i: (i, 0)),
            pl.BlockSpec((1, d), lambda i: (0, 0)),
        ],
        out_specs=pl.BlockSpec((block, d), lambda i: (i, 0)),
        out_shape=jax.ShapeDtypeStruct((n, d), jnp.float32),
    )(feat, m2, tok)


def _pad16(idx):
    n = idx.shape[0]
    p = (-n) % L
    if p:
        idx = jnp.concatenate([idx, jnp.full((p,), _SENTINEL, jnp.int32)])
    return idx


def kernel(feat0, feat1, token0, token1, mask_nodes0, keep_nodes0,
           mask_nodes1, keep_nodes1):
    m0, m1 = _mask_build_kernel()(_pad16(mask_nodes0), _pad16(mask_nodes1))
    masked0 = _blend(feat0, m0.reshape(-1, 1), token0, 10000)
    masked1 = _blend(feat1, m1.reshape(-1, 1), token1, 10000)
    return (masked0, masked1, mask_nodes0, keep_nodes0, mask_nodes1, keep_nodes1)


# speedup vs baseline: 9.7399x; 1.4840x over previous
import functools

import jax
import jax.numpy as jnp
from jax import lax
from jax.experimental import pallas as pl
from jax.experimental.pallas import tpu as pltpu
from jax.experimental.pallas import tpu_sc as plsc

N0, D0 = 100000, 256
N1, D1 = 50000, 128
NM0 = N0 // 10
NM1 = N1 // 10

NS, L = 16, 16
NW = NS

C1 = ((-(-N1 // NW) + L - 1) // L) * L
NM1P = ((NM1 + L - 1) // L) * L

CH = 64
K0 = 10

_SENTINEL = 1 << 30


@functools.lru_cache(maxsize=None)
def _mask1_kernel():
    mesh = plsc.VectorSubcoreMesh(
        core_axis_name="c", subcore_axis_name="s", num_cores=1, num_subcores=NS
    )
    return pl.kernel(
        _mask1_body,
        out_type=jax.ShapeDtypeStruct((NW, C1), jnp.float32),
        mesh=mesh,
        compiler_params=pltpu.CompilerParams(needs_layout_passes=False),
        scratch_types=[
            pltpu.VMEM((NM1P,), jnp.int32),
            pltpu.VMEM((C1,), jnp.float32),
        ],
    )


def _mask1_body(idx_hbm, m_hbm, idx_v, buf):
    wid = lax.axis_index("s")
    zeros = jnp.zeros((L,), jnp.float32)
    ones = jnp.ones((L,), jnp.float32)

    def zero_body(i, carry):
        buf[pl.ds(i * L, L)] = zeros
        return carry

    lax.fori_loop(0, C1 // L, zero_body, 0)

    pltpu.sync_copy(idx_hbm, idx_v)

    base = (wid * C1).astype(jnp.int32)

    def scan_body(j, carry):
        iv = idx_v[pl.ds(j * L, L)]
        rel = iv - base
        msk = (rel >= 0) & (rel < C1)
        safe = jnp.where(msk, rel, 0)
        plsc.store_scatter(buf, [safe], ones, mask=msk)
        return carry

    lax.fori_loop(0, NM1P // L, scan_body, 0)

    pltpu.sync_copy(buf, m_hbm.at[wid])


@functools.lru_cache(maxsize=None)
def _fixup0_kernel():
    mesh = plsc.VectorSubcoreMesh(
        core_axis_name="c", subcore_axis_name="s", num_cores=1, num_subcores=NS
    )
    return pl.kernel(
        _fixup0_body,
        out_type=(),
        mesh=mesh,
        compiler_params=pltpu.CompilerParams(needs_layout_passes=False),
        scratch_types=[
            pltpu.VMEM((K0, CH), jnp.int32),
            pltpu.VMEM((CH, D0), jnp.float32),
            pltpu.SemaphoreType.DMA,
        ],
    )


def _fixup0_body(idx_hbm, tok_hbm, out_hbm, idx_v, rows_v, sem):
    w = lax.axis_index("s")
    pltpu.sync_copy(idx_hbm.at[w], idx_v)
    pltpu.sync_copy(tok_hbm, rows_v.at[pl.ds(0, 1)])
    row0 = tuple(rows_v[0, pl.ds(c * L, L)] for c in range(D0 // L))

    def rep_body(r, carry):
        for c in range(D0 // L):
            rows_v[r, pl.ds(c * L, L)] = row0[c]
        return carry

    lax.fori_loop(1, CH, rep_body, 0)

    copies = [
        pltpu.async_copy(rows_v, out_hbm.at[idx_v.at[j]], sem) for j in range(K0)
    ]
    for cp in copies:
        cp.wait()


def _copy_body(feat_ref, out_ref):
    out_ref[...] = feat_ref[...]


def _copy(feat, block):
    n, d = feat.shape
    return pl.pallas_call(
        _copy_body,
        grid=(n // block,),
        in_specs=[pl.BlockSpec((block, d), lambda i: (i, 0))],
        out_specs=pl.BlockSpec((block, d), lambda i: (i, 0)),
        out_shape=jax.ShapeDtypeStruct((n, d), jnp.float32),
    )(feat)


def _blend_body(feat_ref, m_ref, tok_ref, out_ref):
    out_ref[...] = jnp.where(m_ref[...] > 0.0, tok_ref[...], feat_ref[...])


def _blend(feat, m2, tok, block):
    n, d = feat.shape
    return pl.pallas_call(
        _blend_body,
        grid=(n // block,),
        in_specs=[
            pl.BlockSpec((block, d), lambda i: (i, 0)),
            pl.BlockSpec((block, 1), lambda i: (i, 0)),
            pl.BlockSpec((1, d), lambda i: (0, 0)),
        ],
        out_specs=pl.BlockSpec((block, d), lambda i: (i, 0)),
        out_shape=jax.ShapeDtypeStruct((n, d), jnp.float32),
    )(feat, m2, tok)


def kernel(feat0, feat1, token0, token1, mask_nodes0, keep_nodes0,
           mask_nodes1, keep_nodes1):
    pad0 = NW * K0 * CH - NM0
    idx0_3d = jnp.concatenate(
        [mask_nodes0, jnp.broadcast_to(mask_nodes0[:1], (pad0,))]
    ).reshape(NW, K0, CH)
    idx1_p = jnp.concatenate(
        [mask_nodes1, jnp.full((NM1P - NM1,), _SENTINEL, jnp.int32)]
    )

    out0 = _copy(feat0, 10000)
    m1 = _mask1_kernel()(idx1_p)
    masked1 = _blend(feat1, m1.reshape(-1, 1), token1, 10000)
    ref0 = jax.new_ref(out0)
    _fixup0_kernel()(idx0_3d, token0, ref0)
    masked0 = ref0[...]
    return (masked0, masked1, mask_nodes0, keep_nodes0, mask_nodes1, keep_nodes1)


# speedup vs baseline: 9.7444x; 1.0005x over previous
import functools

import jax
import jax.numpy as jnp
from jax import lax
from jax.experimental import pallas as pl
from jax.experimental.pallas import tpu as pltpu
from jax.experimental.pallas import tpu_sc as plsc

N0, D0 = 100000, 256
N1, D1 = 50000, 128
NM0 = N0 // 10
NM1 = N1 // 10

NS, L = 16, 16
NW = NS

C1 = ((-(-N1 // NW) + L - 1) // L) * L
NM1P = ((NM1 + L - 1) // L) * L

CH = 64
K0 = 10

_SENTINEL = 1 << 30


@functools.lru_cache(maxsize=None)
def _mask1_kernel():
    mesh = plsc.VectorSubcoreMesh(
        core_axis_name="c", subcore_axis_name="s", num_cores=1, num_subcores=NS
    )
    return pl.kernel(
        _mask1_body,
        out_type=jax.ShapeDtypeStruct((NW, C1), jnp.float32),
        mesh=mesh,
        compiler_params=pltpu.CompilerParams(needs_layout_passes=False),
        scratch_types=[
            pltpu.VMEM((NM1P,), jnp.int32),
            pltpu.VMEM((C1,), jnp.float32),
        ],
    )


def _mask1_body(idx_hbm, m_hbm, idx_v, buf):
    wid = lax.axis_index("s")
    zeros = jnp.zeros((L,), jnp.float32)
    ones = jnp.ones((L,), jnp.float32)

    def zero_body(i, carry):
        buf[pl.ds(i * L, L)] = zeros
        return carry

    lax.fori_loop(0, C1 // L, zero_body, 0)

    pltpu.sync_copy(idx_hbm, idx_v)

    base = (wid * C1).astype(jnp.int32)

    def scan_body(j, carry):
        iv = idx_v[pl.ds(j * L, L)]
        rel = iv - base
        msk = (rel >= 0) & (rel < C1)
        safe = jnp.where(msk, rel, 0)
        plsc.store_scatter(buf, [safe], ones, mask=msk)
        return carry

    lax.fori_loop(0, NM1P // L, scan_body, 0)

    pltpu.sync_copy(buf, m_hbm.at[wid])


@functools.lru_cache(maxsize=None)
def _fixup0_kernel():
    mesh = plsc.VectorSubcoreMesh(
        core_axis_name="c", subcore_axis_name="s", num_cores=1, num_subcores=NS
    )
    return pl.kernel(
        _fixup0_body,
        out_type=(),
        mesh=mesh,
        compiler_params=pltpu.CompilerParams(needs_layout_passes=False),
        scratch_types=[
            pltpu.VMEM((K0, CH), jnp.int32),
            pltpu.VMEM((CH, D0), jnp.float32),
            pltpu.SemaphoreType.DMA,
        ],
    )


def _fixup0_body(idx_hbm, tok_hbm, out_hbm, idx_v, rows_v, sem):
    w = lax.axis_index("s")
    pltpu.sync_copy(idx_hbm.at[w], idx_v)
    pltpu.sync_copy(tok_hbm, rows_v.at[pl.ds(0, 1)])
    row0 = tuple(rows_v[0, pl.ds(c * L, L)] for c in range(D0 // L))

    def rep_body(r, carry):
        for c in range(D0 // L):
            rows_v[r, pl.ds(c * L, L)] = row0[c]
        return carry

    lax.fori_loop(1, CH, rep_body, 0)

    copies = [
        pltpu.async_copy(rows_v, out_hbm.at[idx_v.at[j]], sem) for j in range(K0)
    ]
    for cp in copies:
        cp.wait()


def _copy_body(feat_ref, out_ref):
    out_ref[...] = feat_ref[...]


def _copy(feat, block):
    n, d = feat.shape
    return pl.pallas_call(
        _copy_body,
        grid=(n // block,),
        in_specs=[pl.BlockSpec((block, d), lambda i: (i, 0))],
        out_specs=pl.BlockSpec((block, d), lambda i: (i, 0)),
        out_shape=jax.ShapeDtypeStruct((n, d), jnp.float32),
        )(feat)


def _blend_body(feat_ref, m_ref, tok_ref, out_ref):
    out_ref[...] = jnp.where(m_ref[...] > 0.0, tok_ref[...], feat_ref[...])


def _blend(feat, m2, tok, block):
    n, d = feat.shape
    return pl.pallas_call(
        _blend_body,
        grid=(n // block,),
        in_specs=[
            pl.BlockSpec((block, d), lambda i: (i, 0)),
            pl.BlockSpec((block, 1), lambda i: (i, 0)),
            pl.BlockSpec((1, d), lambda i: (0, 0)),
        ],
        out_specs=pl.BlockSpec((block, d), lambda i: (i, 0)),
        out_shape=jax.ShapeDtypeStruct((n, d), jnp.float32),
        )(feat, m2, tok)


def kernel(feat0, feat1, token0, token1, mask_nodes0, keep_nodes0,
           mask_nodes1, keep_nodes1):
    pad0 = NW * K0 * CH - NM0
    idx0_3d = jnp.concatenate(
        [mask_nodes0, jnp.broadcast_to(mask_nodes0[:1], (pad0,))]
    ).reshape(NW, K0, CH)
    idx1_p = jnp.concatenate(
        [mask_nodes1, jnp.full((NM1P - NM1,), _SENTINEL, jnp.int32)]
    )

    m1 = _mask1_kernel()(idx1_p)
    out0 = _copy(feat0, 10000)
    ref0 = jax.new_ref(out0)
    _fixup0_kernel()(idx0_3d, token0, ref0)
    masked1 = _blend(feat1, m1.reshape(-1, 1), token1, 10000)
    masked0 = ref0[...]
    return (masked0, masked1, mask_nodes0, keep_nodes0, mask_nodes1, keep_nodes1)
